# baseline (speedup 1.0000x reference)
import functools

import jax
import jax.numpy as jnp
from jax import lax
from jax.experimental import pallas as pl
from jax.experimental.pallas import tpu as pltpu
from jax.experimental.pallas import tpu_sc as plsc

B, L, D = 4096, 50, 64
BL = B * L
NC, NS = 2, 16
NW = NC * NS
PER_W = BL // NW
CH = 128
NCH = PER_W // CH

_MESH = plsc.VectorSubcoreMesh(core_axis_name="c", subcore_axis_name="s")


@functools.partial(
    pl.kernel,
    mesh=_MESH,
    compiler_params=pltpu.CompilerParams(use_tc_tiling_on_sc=False),
    out_type=jax.ShapeDtypeStruct((BL, D), jnp.float32),
    scratch_types=[
        pltpu.VMEM((NCH, CH), jnp.int32),
        pltpu.VMEM((2, CH, D), jnp.float32),
        pltpu.SemaphoreType.DMA,
        pltpu.SemaphoreType.DMA,
    ],
)
def _gather_sc(x_hbm, table_hbm, out_hbm, idx_v, rows_v, sem0, sem1):
    wid = lax.axis_index("s") * NC + lax.axis_index("c")
    base = wid * PER_W
    pltpu.sync_copy(x_hbm.at[wid], idx_v)
    sems = (sem0, sem1)
    pltpu.async_copy(table_hbm.at[idx_v.at[0]], rows_v.at[0], sem0)
    pltpu.async_copy(table_hbm.at[idx_v.at[1]], rows_v.at[1], sem1)

    def body(i, carry):
        for b in range(2):
            j = 2 * i + b
            pltpu.make_async_copy(
                table_hbm.at[idx_v.at[j]], rows_v.at[b], sems[b]
            ).wait()
            pltpu.sync_copy(rows_v.at[b], out_hbm.at[pl.ds(base + j * CH, CH)])

            @pl.when(j + 2 < NCH)
            def _():
                pltpu.async_copy(
                    table_hbm.at[idx_v.at[j + 2]], rows_v.at[b], sems[b]
                )

        return carry

    lax.fori_loop(0, NCH // 2, body, 0)


NB = 64


def _tr_body(g_ref, o_ref):
    o_ref[...] = jnp.swapaxes(g_ref[...], 1, 2)


def _transpose_tc(g3):
    return pl.pallas_call(
        _tr_body,
        grid=(B // NB,),
        in_specs=[pl.BlockSpec((NB, L, D), lambda i: (i, 0, 0))],
        out_specs=pl.BlockSpec((NB, D, L), lambda i: (i, 0, 0)),
        out_shape=jax.ShapeDtypeStruct((B, D, L), jnp.float32),
    )(g3)


def kernel(x, table):
    xf = x.reshape(-1).astype(jnp.int32).reshape(NW, NCH, CH)
    g = _gather_sc(xf, table)
    return _transpose_tc(g.reshape(B, L, D))
